# baseline (speedup 1.0000x reference)
import functools

import jax
import jax.numpy as jnp
from jax import lax
from jax.experimental import pallas as pl
from jax.experimental.pallas import tpu as pltpu
from jax.experimental.pallas import tpu_sc as plsc

B = 1024
K = 128
EMB = 256
ACT = 2 * EMB
NRELPAD = 512
HUGE = 1e9
EPS = 1e-20
NC, NS, L = 2, 16, 16
NW = NC * NS
RPW = B // NW
DG = EMB // L
KG = K // L

_MESH = plsc.VectorSubcoreMesh(
    core_axis_name="c", subcore_axis_name="s", num_cores=NC, num_subcores=NS)
_SC_PARAMS = pltpu.CompilerParams(needs_layout_passes=False)


@functools.partial(
    pl.kernel,
    out_type=(jax.ShapeDtypeStruct((B, EMB), jnp.float32),
              jax.ShapeDtypeStruct((B, EMB), jnp.float32)),
    mesh=_MESH,
    scratch_types=[
        pltpu.VMEM((RPW,), jnp.int32),
        pltpu.VMEM((RPW,), jnp.int32),
        pltpu.VMEM((RPW, EMB), jnp.float32),
        pltpu.VMEM((RPW, EMB), jnp.float32),
        pltpu.SemaphoreType.DMA,
        pltpu.SemaphoreType.DMA,
    ],
    compiler_params=_SC_PARAMS,
)
def _gather_eq(ent_hbm, rel_hbm, e_hbm, q_hbm, eout_hbm, qout_hbm,
               eidx, qidx, erows, qrows, sem0, sem1):
    wid = lax.axis_index("s") * NC + lax.axis_index("c")
    base = wid * RPW
    pltpu.sync_copy(e_hbm.at[pl.ds(base, RPW)], eidx)
    pltpu.sync_copy(q_hbm.at[pl.ds(base, RPW)], qidx)
    c0 = pltpu.make_async_copy(ent_hbm.at[eidx], erows, sem0)
    c1 = pltpu.make_async_copy(rel_hbm.at[qidx], qrows, sem1)
    c0.start()
    c1.start()
    c0.wait()
    c1.wait()
    pltpu.sync_copy(erows, eout_hbm.at[pl.ds(base, RPW)])
    pltpu.sync_copy(qrows, qout_hbm.at[pl.ds(base, RPW)])


def _dense_body(e_ref, q_ref, h_ref, w1_ref, b1_ref, w2_ref, b2_ref, relp_ref,
                x2e_ref, srel_ref):
    w1 = w1_ref[...]
    x = jnp.dot(e_ref[...], w1[0:EMB], preferred_element_type=jnp.float32)
    x = x + jnp.dot(h_ref[...], w1[EMB:2 * EMB],
                    preferred_element_type=jnp.float32)
    x = x + jnp.dot(q_ref[...], w1[2 * EMB:],
                    preferred_element_type=jnp.float32)
    x = jnp.maximum(x + b1_ref[...], 0.0)
    x2 = jnp.dot(x, w2_ref[...], preferred_element_type=jnp.float32)
    x2 = x2 + b2_ref[...]
    x2e_ref[...] = x2[:, EMB:]
    srel_ref[...] = lax.dot_general(
        x2[:, :EMB], relp_ref[...], (((1,), (1,)), ((), ())),
        preferred_element_type=jnp.float32)


_dense = pl.pallas_call(
    _dense_body,
    out_shape=(jax.ShapeDtypeStruct((B, EMB), jnp.float32),
               jax.ShapeDtypeStruct((B, NRELPAD), jnp.float32)),
)


CAPIDX = K + 2 * L


@functools.partial(
    pl.kernel,
    out_type=jax.ShapeDtypeStruct((NW, RPW * K), jnp.float32),
    mesh=_MESH,
    scratch_types=[
        pltpu.VMEM((RPW * K,), jnp.int32),
        pltpu.VMEM((RPW * K,), jnp.int32),
        pltpu.VMEM((RPW * K,), jnp.int32),
        pltpu.VMEM((RPW * EMB,), jnp.float32),
        pltpu.VMEM((RPW * NRELPAD,), jnp.float32),
        pltpu.VMEM((2 * CAPIDX,), jnp.int32),
        pltpu.VMEM((2 * CAPIDX,), jnp.int32),
        pltpu.VMEM((K, EMB), jnp.float32),
        pltpu.VMEM((K, EMB), jnp.float32),
        pltpu.VMEM((L * L,), jnp.float32),
        pltpu.VMEM((RPW * K,), jnp.float32),
        pltpu.SMEM((4,), jnp.int32),
        pltpu.SemaphoreType.DMA,
        pltpu.SemaphoreType.DMA,
    ],
    compiler_params=_SC_PARAMS,
)
def _score_kernel(ent_hbm, es_hbm, rs_hbm, am_hbm, x2e_hbm, srel_hbm, out_hbm,
                  es_v, idxr_v, mask_v, x2e_v, srel_v, eidx_v, cidx_v,
                  buf0, buf1, scr, scores_v, nk_sm, sem0, sem1):
    wid = lax.axis_index("s") * NC + lax.axis_index("c")
    pltpu.sync_copy(es_hbm.at[wid], es_v)
    pltpu.sync_copy(rs_hbm.at[wid], idxr_v)
    pltpu.sync_copy(am_hbm.at[wid], mask_v)
    pltpu.sync_copy(x2e_hbm.at[wid], x2e_v)
    pltpu.sync_copy(srel_hbm.at[wid], srel_v)

    iota = lax.iota(jnp.int32, L)
    flat_base = iota * L

    def compact(b, slot):
        sbase = slot * CAPIDX

        def cg(kg, off):
            base = b * K + kg * L
            m16i = mask_v[pl.ds(base, L)]
            mb = m16i != 0
            es16 = es_v[pl.ds(base, L)]
            plsc.store_compressed(eidx_v.at[pl.ds(sbase + off, L)], es16,
                                  mask=mb)
            plsc.store_compressed(cidx_v.at[pl.ds(sbase + off, L)],
                                  iota + kg * L, mask=mb)
            ridx = idxr_v[pl.ds(base, L)]
            rel = plsc.load_gather(srel_v, [ridx + b * NRELPAD])
            m16f = m16i.astype(jnp.float32)
            scores_v[pl.ds(base, L)] = rel - (1.0 - m16f) * HUGE
            return off + jnp.sum(m16i)

        off = lax.fori_loop(0, KG, cg, 0)
        zeros16 = jnp.zeros((L,), jnp.int32)
        eidx_v[pl.ds(sbase + off, L)] = zeros16
        eidx_v[pl.ds(sbase + off + L, L)] = zeros16
        cidx_v[pl.ds(sbase + off, L)] = zeros16

        @pl.when(off == 0)
        def _():
            for kg in range(KG):
                eidx_v[pl.ds(sbase + kg * L, L)] = es_v[pl.ds(b * K + kg * L,
                                                              L)]
                cidx_v[pl.ds(sbase + kg * L, L)] = iota + kg * L

        nk = jnp.where(off == 0, K, off)
        nk_sm[2 * slot] = nk
        nk_sm[2 * slot + 1] = (nk + L - 1) // L

    def start(slot, buf, sem):
        sbase = slot * CAPIDX
        nr = (nk_sm[2 * slot] + 31) >> 5
        for v in (1, 2, 3, 4):
            @pl.when(nr == v)
            def _(v=v):
                pltpu.make_async_copy(
                    ent_hbm.at[eidx_v.at[pl.ds(sbase, 32 * v)]],
                    buf.at[pl.ds(0, 32 * v)], sem).start()

    def drain(slot, buf, sem):
        nr = (nk_sm[2 * slot] + 31) >> 5
        for v in (1, 2, 3, 4):
            @pl.when(nr == v)
            def _(v=v):
                pltpu.make_async_copy(
                    ent_hbm.at[eidx_v.at[pl.ds(slot * CAPIDX, 32 * v)]],
                    buf.at[pl.ds(0, 32 * v)], sem).wait()

    def compute_chunks(b, slot, buf):
        sbase = slot * CAPIDX
        xs = [x2e_v[pl.ds(b * EMB + dg * L, L)] for dg in range(DG)]
        nk = nk_sm[2 * slot]

        def cb(j, _):
            for kk in range(L):
                krow = j * L + kk
                prods = [xs[dg] * buf[krow, pl.ds(dg * L, L)]
                         for dg in range(DG)]
                while len(prods) > 1:
                    prods = [a + c for a, c in zip(prods[0::2], prods[1::2])]
                scr[pl.ds(kk * L, L)] = prods[0]
            gs = [plsc.load_gather(scr, [flat_base + d]) for d in range(L)]
            while len(gs) > 1:
                gs = [a + c for a, c in zip(gs[0::2], gs[1::2])]
            cidx16 = cidx_v[pl.ds(sbase + j * L, L)]
            lanemask = (j * L + iota) < nk
            plsc.addupdate_scatter(scores_v, [cidx16 + b * K], gs[0],
                                   mask=lanemask)
            return 0

        lax.fori_loop(0, nk_sm[2 * slot + 1], cb, 0)

    compact(0, 0)
    start(0, buf0, sem0)

    def loop_body(i, _):
        b0 = 2 * i
        compact(b0 + 1, 1)
        start(1, buf1, sem1)
        drain(0, buf0, sem0)
        compute_chunks(b0, 0, buf0)

        @pl.when(b0 + 2 < RPW)
        def _():
            compact(b0 + 2, 0)
            start(0, buf0, sem0)

        drain(1, buf1, sem1)
        compute_chunks(b0 + 1, 1, buf1)
        return 0

    lax.fori_loop(0, RPW // 2, loop_body, 0)
    pltpu.sync_copy(scores_v, out_hbm.at[wid])


def _soft_body(s_ref, p_ref, ent_ref):
    s = s_ref[...]
    m = jnp.max(s, axis=-1, keepdims=True)
    ex = jnp.exp(s - m)
    z = jnp.sum(ex, axis=-1, keepdims=True)
    p = ex / z
    p_ref[...] = p
    ent_ref[...] = -jnp.sum(p * jnp.log(p + EPS), axis=-1, keepdims=True)


_soft = pl.pallas_call(
    _soft_body,
    out_shape=(jax.ShapeDtypeStruct((B, K), jnp.float32),
               jax.ShapeDtypeStruct((B, 1), jnp.float32)),
)


def kernel(e, q, H, r_space, e_space, action_mask,
           entity_table, relation_table, W1, b1, W2, b2):
    e = e.astype(jnp.int32)
    q = q.astype(jnp.int32)
    E, Q = _gather_eq(entity_table, relation_table, e, q)
    relp = jnp.pad(relation_table,
                   ((0, NRELPAD - relation_table.shape[0]), (0, 0)))
    x2e, srel = _dense(E, Q, H, W1, b1.reshape(1, ACT), W2,
                       b2.reshape(1, ACT), relp)
    scores = _score_kernel(
        entity_table,
        e_space.astype(jnp.int32).reshape(NW, RPW * K),
        r_space.astype(jnp.int32).reshape(NW, RPW * K),
        action_mask.astype(jnp.int32).reshape(NW, RPW * K),
        x2e.reshape(NW, RPW * EMB),
        srel.reshape(NW, RPW * NRELPAD),
    )
    p, ent = _soft(scores.reshape(B, K))
    return p, ent.reshape(B)


# speedup vs baseline: 5.9897x; 5.9897x over previous
import functools

import jax
import jax.numpy as jnp
from jax import lax
from jax.experimental import pallas as pl
from jax.experimental.pallas import tpu as pltpu
from jax.experimental.pallas import tpu_sc as plsc

B = 1024
K = 128
EMB = 256
ACT = 2 * EMB
NRELPAD = 512
HUGE = 1e9
EPS = 1e-20
NC, NS, L = 2, 16, 16
NW = NC * NS
RPW = B // NW
DG = EMB // L
KG = K // L

_MESH = plsc.VectorSubcoreMesh(
    core_axis_name="c", subcore_axis_name="s", num_cores=NC, num_subcores=NS)
_SC_PARAMS = pltpu.CompilerParams(needs_layout_passes=False)


@functools.partial(
    pl.kernel,
    out_type=(jax.ShapeDtypeStruct((B, EMB), jnp.float32),
              jax.ShapeDtypeStruct((B, EMB), jnp.float32)),
    mesh=_MESH,
    scratch_types=[
        pltpu.VMEM((RPW,), jnp.int32),
        pltpu.VMEM((RPW,), jnp.int32),
        pltpu.VMEM((RPW, EMB), jnp.float32),
        pltpu.VMEM((RPW, EMB), jnp.float32),
        pltpu.SemaphoreType.DMA,
        pltpu.SemaphoreType.DMA,
    ],
    compiler_params=_SC_PARAMS,
)
def _gather_eq(ent_hbm, rel_hbm, e_hbm, q_hbm, eout_hbm, qout_hbm,
               eidx, qidx, erows, qrows, sem0, sem1):
    wid = lax.axis_index("s") * NC + lax.axis_index("c")
    base = wid * RPW
    pltpu.sync_copy(e_hbm.at[pl.ds(base, RPW)], eidx)
    pltpu.sync_copy(q_hbm.at[pl.ds(base, RPW)], qidx)
    c0 = pltpu.make_async_copy(ent_hbm.at[eidx], erows, sem0)
    c1 = pltpu.make_async_copy(rel_hbm.at[qidx], qrows, sem1)
    c0.start()
    c1.start()
    c0.wait()
    c1.wait()
    pltpu.sync_copy(erows, eout_hbm.at[pl.ds(base, RPW)])
    pltpu.sync_copy(qrows, qout_hbm.at[pl.ds(base, RPW)])


def _dense_body(e_ref, q_ref, h_ref, w1_ref, b1_ref, w2_ref, b2_ref, relp_ref,
                x2e_ref, srel_ref):
    w1 = w1_ref[...]
    x = jnp.dot(e_ref[...], w1[0:EMB], preferred_element_type=jnp.float32)
    x = x + jnp.dot(h_ref[...], w1[EMB:2 * EMB],
                    preferred_element_type=jnp.float32)
    x = x + jnp.dot(q_ref[...], w1[2 * EMB:],
                    preferred_element_type=jnp.float32)
    x = jnp.maximum(x + b1_ref[...], 0.0)
    x2 = jnp.dot(x, w2_ref[...], preferred_element_type=jnp.float32)
    x2 = x2 + b2_ref[...]
    x2e_ref[...] = x2[:, EMB:]
    srel_ref[...] = lax.dot_general(
        x2[:, :EMB], relp_ref[...], (((1,), (1,)), ((), ())),
        preferred_element_type=jnp.float32)


_dense = pl.pallas_call(
    _dense_body,
    out_shape=(jax.ShapeDtypeStruct((B, EMB), jnp.float32),
               jax.ShapeDtypeStruct((B, NRELPAD), jnp.float32)),
)


CAPIDX = K + L


@functools.partial(
    pl.kernel,
    out_type=jax.ShapeDtypeStruct((NW, RPW * K), jnp.float32),
    mesh=_MESH,
    scratch_types=[
        pltpu.VMEM((RPW * K,), jnp.int32),
        pltpu.VMEM((RPW * K,), jnp.int32),
        pltpu.VMEM((RPW * K,), jnp.int32),
        pltpu.VMEM((RPW * EMB,), jnp.float32),
        pltpu.VMEM((RPW * NRELPAD,), jnp.float32),
        pltpu.VMEM((2 * CAPIDX,), jnp.int32),
        pltpu.VMEM((K, EMB), jnp.float32),
        pltpu.VMEM((K, EMB), jnp.float32),
        pltpu.VMEM((L * L,), jnp.float32),
        pltpu.VMEM((RPW * K,), jnp.float32),
        pltpu.SMEM((4,), jnp.int32),
        pltpu.SemaphoreType.DMA,
        pltpu.SemaphoreType.DMA,
    ],
    compiler_params=_SC_PARAMS,
)
def _score_kernel(ent_hbm, es_hbm, rs_hbm, am_hbm, x2e_hbm, srel_hbm, out_hbm,
                  es_v, idxr_v, mask_v, x2e_v, srel_v, cidx_v,
                  buf0, buf1, scr, scores_v, nk_sm, sem0, sem1):
    wid = lax.axis_index("s") * NC + lax.axis_index("c")
    pltpu.sync_copy(es_hbm.at[wid], es_v)
    pltpu.sync_copy(rs_hbm.at[wid], idxr_v)
    pltpu.sync_copy(am_hbm.at[wid], mask_v)
    pltpu.sync_copy(x2e_hbm.at[wid], x2e_v)
    pltpu.sync_copy(srel_hbm.at[wid], srel_v)

    iota = lax.iota(jnp.int32, L)
    flat_base = iota * L

    def compact(b, slot):
        sbase = slot * CAPIDX

        def cg(kg, off):
            base = b * K + kg * L
            m16i = mask_v[pl.ds(base, L)]
            mb = m16i != 0
            plsc.store_compressed(cidx_v.at[pl.ds(sbase + off, L)],
                                  iota + kg * L, mask=mb)
            ridx = idxr_v[pl.ds(base, L)]
            rel = plsc.load_gather(srel_v, [ridx + b * NRELPAD])
            m16f = m16i.astype(jnp.float32)
            scores_v[pl.ds(base, L)] = rel - (1.0 - m16f) * HUGE
            return off + jnp.sum(m16i)

        off = lax.fori_loop(0, KG, cg, 0)
        cidx_v[pl.ds(sbase + off, L)] = jnp.zeros((L,), jnp.int32)

        @pl.when(off == 0)
        def _():
            for kg in range(KG):
                cidx_v[pl.ds(sbase + kg * L, L)] = iota + kg * L

        nk = jnp.where(off == 0, K, off)
        nk_sm[2 * slot] = nk
        nk_sm[2 * slot + 1] = (nk + L - 1) // L

    def start(b, buf, sem):
        pltpu.make_async_copy(ent_hbm.at[es_v.at[pl.ds(b * K, K)]],
                              buf, sem).start()

    def drain(buf, sem):
        pltpu.make_async_copy(ent_hbm.at[es_v.at[pl.ds(0, K)]],
                              buf, sem).wait()

    def compute_chunks(b, slot, buf):
        sbase = slot * CAPIDX
        xs = [x2e_v[pl.ds(b * EMB + dg * L, L)] for dg in range(DG)]
        nk = nk_sm[2 * slot]

        def cb(j, _):
            cidx16 = cidx_v[pl.ds(sbase + j * L, L)]
            for kk in range(L):
                krow = cidx16[kk]
                prods = [xs[dg] * buf[krow, pl.ds(dg * L, L)]
                         for dg in range(DG)]
                while len(prods) > 1:
                    prods = [a + c for a, c in zip(prods[0::2], prods[1::2])]
                scr[pl.ds(kk * L, L)] = prods[0]
            gs = [plsc.load_gather(scr, [flat_base + d]) for d in range(L)]
            while len(gs) > 1:
                gs = [a + c for a, c in zip(gs[0::2], gs[1::2])]
            lanemask = (j * L + iota) < nk
            plsc.addupdate_scatter(scores_v, [cidx16 + b * K], gs[0],
                                   mask=lanemask)
            return 0

        lax.fori_loop(0, nk_sm[2 * slot + 1], cb, 0)

    start(0, buf0, sem0)
    compact(0, 0)

    def loop_body(i, _):
        b0 = 2 * i
        start(b0 + 1, buf1, sem1)
        compact(b0 + 1, 1)
        drain(buf0, sem0)
        compute_chunks(b0, 0, buf0)

        @pl.when(b0 + 2 < RPW)
        def _():
            start(b0 + 2, buf0, sem0)
            compact(b0 + 2, 0)

        drain(buf1, sem1)
        compute_chunks(b0 + 1, 1, buf1)
        return 0

    lax.fori_loop(0, RPW // 2, loop_body, 0)
    pltpu.sync_copy(scores_v, out_hbm.at[wid])


def _soft_body(s_ref, p_ref, ent_ref):
    s = s_ref[...]
    m = jnp.max(s, axis=-1, keepdims=True)
    ex = jnp.exp(s - m)
    z = jnp.sum(ex, axis=-1, keepdims=True)
    p = ex / z
    p_ref[...] = p
    ent_ref[...] = -jnp.sum(p * jnp.log(p + EPS), axis=-1, keepdims=True)


_soft = pl.pallas_call(
    _soft_body,
    out_shape=(jax.ShapeDtypeStruct((B, K), jnp.float32),
               jax.ShapeDtypeStruct((B, 1), jnp.float32)),
)


def kernel(e, q, H, r_space, e_space, action_mask,
           entity_table, relation_table, W1, b1, W2, b2):
    e = e.astype(jnp.int32)
    q = q.astype(jnp.int32)
    E, Q = _gather_eq(entity_table, relation_table, e, q)
    relp = jnp.pad(relation_table,
                   ((0, NRELPAD - relation_table.shape[0]), (0, 0)))
    x2e, srel = _dense(E, Q, H, W1, b1.reshape(1, ACT), W2,
                       b2.reshape(1, ACT), relp)
    scores = _score_kernel(
        entity_table,
        e_space.astype(jnp.int32).reshape(NW, RPW * K),
        r_space.astype(jnp.int32).reshape(NW, RPW * K),
        action_mask.astype(jnp.int32).reshape(NW, RPW * K),
        x2e.reshape(NW, RPW * EMB),
        srel.reshape(NW, RPW * NRELPAD),
    )
    p, ent = _soft(scores.reshape(B, K))
    return p, ent.reshape(B)


# speedup vs baseline: 6.1033x; 1.0190x over previous
import functools

import jax
import jax.numpy as jnp
from jax import lax
from jax.experimental import pallas as pl
from jax.experimental.pallas import tpu as pltpu
from jax.experimental.pallas import tpu_sc as plsc

B = 1024
K = 128
EMB = 256
ACT = 2 * EMB
NRELPAD = 512
HUGE = 1e9
EPS = 1e-20
NC, NS, L = 2, 16, 16
NW = NC * NS
RPW = B // NW
DG = EMB // L
KG = K // L

_MESH = plsc.VectorSubcoreMesh(
    core_axis_name="c", subcore_axis_name="s", num_cores=NC, num_subcores=NS)
_SC_PARAMS = pltpu.CompilerParams(needs_layout_passes=False)


@functools.partial(
    pl.kernel,
    out_type=(jax.ShapeDtypeStruct((B, EMB), jnp.float32),
              jax.ShapeDtypeStruct((B, EMB), jnp.float32)),
    mesh=_MESH,
    scratch_types=[
        pltpu.VMEM((RPW,), jnp.int32),
        pltpu.VMEM((RPW,), jnp.int32),
        pltpu.VMEM((RPW, EMB), jnp.float32),
        pltpu.VMEM((RPW, EMB), jnp.float32),
        pltpu.SemaphoreType.DMA,
        pltpu.SemaphoreType.DMA,
    ],
    compiler_params=_SC_PARAMS,
)
def _gather_eq(ent_hbm, rel_hbm, e_hbm, q_hbm, eout_hbm, qout_hbm,
               eidx, qidx, erows, qrows, sem0, sem1):
    wid = lax.axis_index("s") * NC + lax.axis_index("c")
    base = wid * RPW
    pltpu.sync_copy(e_hbm.at[pl.ds(base, RPW)], eidx)
    pltpu.sync_copy(q_hbm.at[pl.ds(base, RPW)], qidx)
    c0 = pltpu.make_async_copy(ent_hbm.at[eidx], erows, sem0)
    c1 = pltpu.make_async_copy(rel_hbm.at[qidx], qrows, sem1)
    c0.start()
    c1.start()
    c0.wait()
    c1.wait()
    pltpu.sync_copy(erows, eout_hbm.at[pl.ds(base, RPW)])
    pltpu.sync_copy(qrows, qout_hbm.at[pl.ds(base, RPW)])


def _dense_body(e_ref, q_ref, h_ref, w1_ref, b1_ref, w2_ref, b2_ref, relp_ref,
                x2e_ref, srel_ref):
    w1 = w1_ref[...]
    x = jnp.dot(e_ref[...], w1[0:EMB], preferred_element_type=jnp.float32)
    x = x + jnp.dot(h_ref[...], w1[EMB:2 * EMB],
                    preferred_element_type=jnp.float32)
    x = x + jnp.dot(q_ref[...], w1[2 * EMB:],
                    preferred_element_type=jnp.float32)
    x = jnp.maximum(x + b1_ref[...], 0.0)
    x2 = jnp.dot(x, w2_ref[...], preferred_element_type=jnp.float32)
    x2 = x2 + b2_ref[...]
    x2e_ref[...] = x2[:, EMB:]
    srel_ref[...] = lax.dot_general(
        x2[:, :EMB], relp_ref[...], (((1,), (1,)), ((), ())),
        preferred_element_type=jnp.float32)


_dense = pl.pallas_call(
    _dense_body,
    out_shape=(jax.ShapeDtypeStruct((B, EMB), jnp.float32),
               jax.ShapeDtypeStruct((B, NRELPAD), jnp.float32)),
)


CAPIDX = K + L


@functools.partial(
    pl.kernel,
    out_type=jax.ShapeDtypeStruct((NW, RPW * K), jnp.float32),
    mesh=_MESH,
    scratch_types=[
        pltpu.VMEM((RPW * K,), jnp.int32),
        pltpu.VMEM((RPW * K,), jnp.int32),
        pltpu.VMEM((RPW * K,), jnp.int32),
        pltpu.VMEM((RPW * EMB,), jnp.float32),
        pltpu.VMEM((RPW * NRELPAD,), jnp.float32),
        pltpu.VMEM((2 * CAPIDX,), jnp.int32),
        pltpu.VMEM((K, EMB), jnp.float32),
        pltpu.VMEM((K, EMB), jnp.float32),
        pltpu.VMEM((L * L,), jnp.float32),
        pltpu.VMEM((RPW * K,), jnp.float32),
        pltpu.SMEM((4,), jnp.int32),
        pltpu.SemaphoreType.DMA,
        pltpu.SemaphoreType.DMA,
    ],
    compiler_params=_SC_PARAMS,
)
def _score_kernel(ent_hbm, es_hbm, rs_hbm, am_hbm, x2e_hbm, srel_hbm, out_hbm,
                  es_v, idxr_v, mask_v, x2e_v, srel_v, cidx_v,
                  buf0, buf1, scr, scores_v, nk_sm, sem0, sem1):
    wid = lax.axis_index("s") * NC + lax.axis_index("c")
    pltpu.sync_copy(es_hbm.at[wid], es_v)
    pltpu.make_async_copy(ent_hbm.at[es_v.at[pl.ds(0, K)]],
                          buf0, sem0).start()
    pltpu.sync_copy(rs_hbm.at[wid], idxr_v)
    pltpu.sync_copy(am_hbm.at[wid], mask_v)
    pltpu.sync_copy(x2e_hbm.at[wid], x2e_v)
    pltpu.sync_copy(srel_hbm.at[wid], srel_v)

    iota = lax.iota(jnp.int32, L)
    flat_base = iota * L

    def compact(b, slot):
        sbase = slot * CAPIDX

        def cg(kg, off):
            base = b * K + kg * L
            m16i = mask_v[pl.ds(base, L)]
            mb = m16i != 0
            plsc.store_compressed(cidx_v.at[pl.ds(sbase + off, L)],
                                  iota + kg * L, mask=mb)
            ridx = idxr_v[pl.ds(base, L)]
            rel = plsc.load_gather(srel_v, [ridx + b * NRELPAD])
            m16f = m16i.astype(jnp.float32)
            scores_v[pl.ds(base, L)] = rel - (1.0 - m16f) * HUGE
            return off + jnp.sum(m16i)

        off = lax.fori_loop(0, KG, cg, 0)
        cidx_v[pl.ds(sbase + off, L)] = jnp.zeros((L,), jnp.int32)

        @pl.when(off == 0)
        def _():
            for kg in range(KG):
                cidx_v[pl.ds(sbase + kg * L, L)] = iota + kg * L

        nk = jnp.where(off == 0, K, off)
        nk_sm[2 * slot] = nk
        nk_sm[2 * slot + 1] = (nk + L - 1) // L

    def start(b, buf, sem):
        pltpu.make_async_copy(ent_hbm.at[es_v.at[pl.ds(b * K, K)]],
                              buf, sem).start()

    def drain(buf, sem):
        pltpu.make_async_copy(ent_hbm.at[es_v.at[pl.ds(0, K)]],
                              buf, sem).wait()

    def compute_chunks(b, slot, buf):
        sbase = slot * CAPIDX
        xs = [x2e_v[pl.ds(b * EMB + dg * L, L)] for dg in range(DG)]
        nk = nk_sm[2 * slot]

        def cb(j, _):
            cidx16 = cidx_v[pl.ds(sbase + j * L, L)]
            for kk in range(L):
                krow = cidx16[kk]
                prods = [xs[dg] * buf[krow, pl.ds(dg * L, L)]
                         for dg in range(DG)]
                while len(prods) > 1:
                    prods = [a + c for a, c in zip(prods[0::2], prods[1::2])]
                scr[pl.ds(kk * L, L)] = prods[0]
            gs = [plsc.load_gather(scr, [flat_base + d]) for d in range(L)]
            while len(gs) > 1:
                gs = [a + c for a, c in zip(gs[0::2], gs[1::2])]
            lanemask = (j * L + iota) < nk
            plsc.addupdate_scatter(scores_v, [cidx16 + b * K], gs[0],
                                   mask=lanemask)
            return 0

        lax.fori_loop(0, nk_sm[2 * slot + 1], cb, 0)

    compact(0, 0)

    def loop_body(i, _):
        b0 = 2 * i
        start(b0 + 1, buf1, sem1)
        compact(b0 + 1, 1)
        drain(buf0, sem0)
        compute_chunks(b0, 0, buf0)

        @pl.when(b0 + 2 < RPW)
        def _():
            start(b0 + 2, buf0, sem0)
            compact(b0 + 2, 0)

        drain(buf1, sem1)
        compute_chunks(b0 + 1, 1, buf1)
        return 0

    lax.fori_loop(0, RPW // 2, loop_body, 0)
    pltpu.sync_copy(scores_v, out_hbm.at[wid])


def _soft_body(s_ref, p_ref, ent_ref):
    s = s_ref[...]
    m = jnp.max(s, axis=-1, keepdims=True)
    ex = jnp.exp(s - m)
    z = jnp.sum(ex, axis=-1, keepdims=True)
    p = ex / z
    p_ref[...] = p
    ent_ref[...] = -jnp.sum(p * jnp.log(p + EPS), axis=-1, keepdims=True)


_soft = pl.pallas_call(
    _soft_body,
    out_shape=(jax.ShapeDtypeStruct((B, K), jnp.float32),
               jax.ShapeDtypeStruct((B, 1), jnp.float32)),
)


def kernel(e, q, H, r_space, e_space, action_mask,
           entity_table, relation_table, W1, b1, W2, b2):
    e = e.astype(jnp.int32)
    q = q.astype(jnp.int32)
    E, Q = _gather_eq(entity_table, relation_table, e, q)
    relp = jnp.pad(relation_table,
                   ((0, NRELPAD - relation_table.shape[0]), (0, 0)))
    x2e, srel = _dense(E, Q, H, W1, b1.reshape(1, ACT), W2,
                       b2.reshape(1, ACT), relp)
    scores = _score_kernel(
        entity_table,
        e_space.astype(jnp.int32).reshape(NW, RPW * K),
        r_space.astype(jnp.int32).reshape(NW, RPW * K),
        action_mask.astype(jnp.int32).reshape(NW, RPW * K),
        x2e.reshape(NW, RPW * EMB),
        srel.reshape(NW, RPW * NRELPAD),
    )
    p, ent = _soft(scores.reshape(B, K))
    return p, ent.reshape(B)
